# baseline (speedup 1.0000x reference)
import functools

import jax
import jax.numpy as jnp
from jax import lax
from jax.experimental import pallas as pl
from jax.experimental.pallas import tpu as pltpu
from jax.experimental.pallas import tpu_sc as plsc

_LAMB = 0.025
_B = 16384
_D = 64
_NC = 2
_NS = 16
_NW = _NC * _NS
_BPW = _B // _NW
_CH = 128
_NCH = _BPW // _CH
_GRP = _BPW // 16


def _bpr_body(user_hbm, item_i_hbm, item_j_hbm, eu_hbm, ei_hbm,
              pi_hbm, pj_hbm, reg_hbm,
              idx_u, idx_i, idx_j,
              rows_u, rows_i, rows_j,
              out_pi, out_pj, out_reg,
              sem_u, sem_i, sem_j):
  c = lax.axis_index("c")
  s = lax.axis_index("s")
  wid = s * _NC + c
  base = wid * _BPW

  pltpu.sync_copy(user_hbm.at[pl.ds(base, _BPW)], idx_u)
  pltpu.sync_copy(item_i_hbm.at[pl.ds(base, _BPW)], idx_i)
  pltpu.sync_copy(item_j_hbm.at[pl.ds(base, _BPW)], idx_j)

  copies = []
  for k in range(_NCH):
    sl = pl.ds(k * _CH, _CH)
    copies.append(pltpu.async_copy(eu_hbm.at[idx_u.at[sl]], rows_u.at[sl], sem_u))
    copies.append(pltpu.async_copy(ei_hbm.at[idx_i.at[sl]], rows_i.at[sl], sem_i))
    copies.append(pltpu.async_copy(ei_hbm.at[idx_j.at[sl]], rows_j.at[sl], sem_j))
  for cp in copies:
    cp.wait()

  lane = lax.iota(jnp.int32, 16)

  def group(g, carry):
    vals_pi = jnp.zeros((16,), jnp.float32)
    vals_pj = jnp.zeros((16,), jnp.float32)
    vals_rg = jnp.zeros((16,), jnp.float32)
    for l in range(16):
      r = g * 16 + l
      u = [rows_u[r, pl.ds(16 * t, 16)] for t in range(4)]
      iv = [rows_i[r, pl.ds(16 * t, 16)] for t in range(4)]
      jv = [rows_j[r, pl.ds(16 * t, 16)] for t in range(4)]
      pi = u[0] * iv[0] + u[1] * iv[1] + u[2] * iv[2] + u[3] * iv[3]
      pj = u[0] * jv[0] + u[1] * jv[1] + u[2] * jv[2] + u[3] * jv[3]
      rg = (u[0] * u[0] + u[1] * u[1] + u[2] * u[2] + u[3] * u[3]
            + iv[0] * iv[0] + iv[1] * iv[1] + iv[2] * iv[2] + iv[3] * iv[3]
            + jv[0] * jv[0] + jv[1] * jv[1] + jv[2] * jv[2] + jv[3] * jv[3])
      m = lane == l
      vals_pi = jnp.where(m, jnp.sum(pi), vals_pi)
      vals_pj = jnp.where(m, jnp.sum(pj), vals_pj)
      vals_rg = jnp.where(m, jnp.sum(rg), vals_rg)
    sl = pl.ds(g * 16, 16)
    out_pi[sl] = vals_pi
    out_pj[sl] = vals_pj
    out_reg[sl] = vals_rg * _LAMB
    return carry

  lax.fori_loop(0, _GRP, group, None)

  pltpu.sync_copy(out_pi, pi_hbm.at[pl.ds(base, _BPW)])
  pltpu.sync_copy(out_pj, pj_hbm.at[pl.ds(base, _BPW)])
  pltpu.sync_copy(out_reg, reg_hbm.at[pl.ds(base, _BPW)])


@jax.jit
def _bpr(user, item_i, item_j, embed_user, embed_item):
  mesh = plsc.VectorSubcoreMesh(
      core_axis_name="c", subcore_axis_name="s",
      num_cores=_NC, num_subcores=_NS)
  out = jax.ShapeDtypeStruct((_B,), jnp.float32)
  f = pl.kernel(
      _bpr_body,
      out_type=[out, out, out],
      mesh=mesh,
      compiler_params=pltpu.CompilerParams(
          needs_layout_passes=False, use_tc_tiling_on_sc=False),
      scratch_types=[
          pltpu.VMEM((_BPW,), jnp.int32),
          pltpu.VMEM((_BPW,), jnp.int32),
          pltpu.VMEM((_BPW,), jnp.int32),
          pltpu.VMEM((_BPW, _D), jnp.float32),
          pltpu.VMEM((_BPW, _D), jnp.float32),
          pltpu.VMEM((_BPW, _D), jnp.float32),
          pltpu.VMEM((_BPW,), jnp.float32),
          pltpu.VMEM((_BPW,), jnp.float32),
          pltpu.VMEM((_BPW,), jnp.float32),
          pltpu.SemaphoreType.DMA,
          pltpu.SemaphoreType.DMA,
          pltpu.SemaphoreType.DMA,
      ],
  )
  pi, pj, reg = f(user, item_i, item_j, embed_user, embed_item)
  return pi, pj, reg


def kernel(user, item_i, item_j, embed_user, embed_item):
  user = jnp.asarray(user, jnp.int32)
  item_i = jnp.asarray(item_i, jnp.int32)
  item_j = jnp.asarray(item_j, jnp.int32)
  return _bpr(user, item_i, item_j, embed_user, embed_item)
